# Initial kernel scaffold; baseline (speedup 1.0000x reference)
import jax
import jax.numpy as jnp
from jax.experimental import pallas as pl


def kernel(Xh, Xd, Xw, A, WgH1, bgH1, wcH1, bcH1, WgH2, bgH2, wcH2, bcH2, WgD1, bgD1, wcD1, bcD1, WgD2, bgD2, wcD2, bcD2, WgW1, bgW1, wcW1, bcW1, WgW2, bgW2, wcW2, bcW2, WlD, blD, Wh, Wd, Ww):
    raise NotImplementedError("write your pallas kernel here")


# speedup vs baseline: 1.6424x; 1.6424x over previous
import jax
import jax.numpy as jnp
from jax import lax
from jax.experimental import pallas as pl
from jax.experimental.pallas import tpu as pltpu

_N = 2048
_B = 4
_TILE = 256
_GRID = _N // _TILE


def _rowstats_body(a_ref, dinv_ref, adiag_ref):
    i = pl.program_id(0)
    a = a_ref[...]
    rowsum = jnp.sum(a, axis=1)
    row = lax.broadcasted_iota(jnp.int32, a.shape, 0)
    col = lax.broadcasted_iota(jnp.int32, a.shape, 1)
    diag = jnp.sum(jnp.where(col == row + i * _TILE, a, 0.0), axis=1)
    deg = rowsum - diag
    pos = deg > 0.0
    dinv = jnp.where(pos, lax.rsqrt(jnp.where(pos, deg, 1.0)), 0.0)
    dinv_ref[...] = dinv[:, None]
    adiag_ref[...] = diag[:, None]


def _rowstats(A):
    return pl.pallas_call(
        _rowstats_body,
        grid=(_GRID,),
        in_specs=[pl.BlockSpec((_TILE, _N), lambda i: (i, 0))],
        out_specs=(pl.BlockSpec((_TILE, 1), lambda i: (i, 0)),
                   pl.BlockSpec((_TILE, 1), lambda i: (i, 0))),
        out_shape=(jax.ShapeDtypeStruct((_N, 1), jnp.float32),
                   jax.ShapeDtypeStruct((_N, 1), jnp.float32)),
    )(A)


def _lhop1_body(x_ref, dinv_ref, adiag_ref, a_ref, out_ref, y_ref):
    i = pl.program_id(0)

    @pl.when(i == 0)
    def _():
        y_ref[...] = x_ref[...] * dinv_ref[...]

    acc = jnp.dot(a_ref[...], y_ref[...], preferred_element_type=jnp.float32)
    sl = pl.ds(i * _TILE, _TILE)
    out_ref[...] = -dinv_ref[sl, :] * (acc - adiag_ref[sl, :] * y_ref[sl, :])


def _lhop1(x, dinv, adiag, A):
    w = x.shape[1]
    return pl.pallas_call(
        _lhop1_body,
        grid=(_GRID,),
        in_specs=[
            pl.BlockSpec((_N, w), lambda i: (0, 0)),
            pl.BlockSpec((_N, 1), lambda i: (0, 0)),
            pl.BlockSpec((_N, 1), lambda i: (0, 0)),
            pl.BlockSpec((_TILE, _N), lambda i: (i, 0)),
        ],
        out_specs=pl.BlockSpec((_TILE, w), lambda i: (i, 0)),
        out_shape=jax.ShapeDtypeStruct((_N, w), jnp.float32),
        scratch_shapes=[pltpu.VMEM((_N, w), jnp.float32)],
    )(x, dinv, adiag, A)


def _tconv_relu(x, wc_ref, group):
    t, _ = x.shape
    z = jnp.zeros((t, 1), dtype=x.dtype)
    xl = jnp.concatenate([z, x[:, :-1]], axis=1)
    xr = jnp.concatenate([x[:, 1:], z], axis=1)
    colm = lax.broadcasted_iota(jnp.int32, x.shape, 1) % group
    xl = jnp.where(colm == 0, 0.0, xl)
    xr = jnp.where(colm == group - 1, 0.0, xr)
    y = (wc_ref[:, 0:1] * xl + wc_ref[:, 1:2] * x + wc_ref[:, 2:3] * xr
         + wc_ref[:, 3:4])
    return jnp.maximum(y, 0.0)


_B1_SLICES = ((0, 96), (96, 144), (144, 240))
_B2_SLICES = ((0, 256), (256, 512), (512, 768))


def _block1_hop2_body(tx1_ref, x_ref, dinv_ref, adiag_ref, a_ref,
                      wh_ref, wd_ref, ww_ref, bh_ref, bd_ref, bw_ref,
                      ch_ref, cd_ref, cw_ref, out_ref, y_ref):
    i = pl.program_id(0)

    @pl.when(i == 0)
    def _():
        y_ref[...] = tx1_ref[...] * dinv_ref[...]

    acc = jnp.dot(a_ref[...], y_ref[...], preferred_element_type=jnp.float32)
    sl = pl.ds(i * _TILE, _TILE)
    ltx1 = -dinv_ref[sl, :] * (acc - adiag_ref[sl, :] * y_ref[sl, :])
    tx0 = x_ref[...]
    tx1 = tx1_ref[sl, :]
    tx2 = 2.0 * ltx1 - tx0

    outs = []
    for (lo, hi), w_ref, b_ref, c_ref in (
        (_B1_SLICES[0], wh_ref, bh_ref, ch_ref),
        (_B1_SLICES[1], wd_ref, bd_ref, cd_ref),
        (_B1_SLICES[2], ww_ref, bw_ref, cw_ref),
    ):
        o = (jnp.dot(tx0[:, lo:hi], w_ref[0], preferred_element_type=jnp.float32)
             + jnp.dot(tx1[:, lo:hi], w_ref[1], preferred_element_type=jnp.float32)
             + jnp.dot(tx2[:, lo:hi], w_ref[2], preferred_element_type=jnp.float32))
        o = jnp.maximum(o + b_ref[...], 0.0)
        outs.append(_tconv_relu(o, c_ref, 64))
    out_ref[...] = jnp.concatenate(outs, axis=1)


def _block1_hop2(tx1, x, dinv, adiag, A, wh, wd, ww, bh, bd, bw, ch, cd, cw):
    full = lambda s: pl.BlockSpec(s, lambda i: (0,) * len(s))
    return pl.pallas_call(
        _block1_hop2_body,
        grid=(_GRID,),
        in_specs=[
            full((_N, 240)),
            pl.BlockSpec((_TILE, 240), lambda i: (i, 0)),
            full((_N, 1)),
            full((_N, 1)),
            pl.BlockSpec((_TILE, _N), lambda i: (i, 0)),
            full((3, 96, 256)), full((3, 48, 256)), full((3, 96, 256)),
            full((1, 256)), full((1, 256)), full((1, 256)),
            full((1, 4)), full((1, 4)), full((1, 4)),
        ],
        out_specs=pl.BlockSpec((_TILE, 768), lambda i: (i, 0)),
        out_shape=jax.ShapeDtypeStruct((_N, 768), jnp.float32),
        scratch_shapes=[pltpu.VMEM((_N, 240), jnp.float32)],
    )(tx1, x, dinv, adiag, A, wh, wd, ww, bh, bd, bw, ch, cd, cw)


def _block2_hop2_body(tx1_ref, y1_ref, dinv_ref, adiag_ref, a_ref,
                      wh_ref, wd_ref, ww_ref, bh_ref, bd_ref, bw_ref,
                      ch_ref, cd_ref, cw_ref,
                      wl_ref, bl_ref, sh_ref, sd_ref, sw_ref,
                      out_ref, y_ref):
    i = pl.program_id(0)

    @pl.when(i == 0)
    def _():
        y_ref[...] = tx1_ref[...] * dinv_ref[...]

    acc = jnp.dot(a_ref[...], y_ref[...], preferred_element_type=jnp.float32)
    sl = pl.ds(i * _TILE, _TILE)
    ltx1 = -dinv_ref[sl, :] * (acc - adiag_ref[sl, :] * y_ref[sl, :])
    tx0 = y1_ref[...]
    tx1 = tx1_ref[sl, :]
    tx2 = 2.0 * ltx1 - tx0

    res = jnp.zeros((_TILE, _B * 12), jnp.float32)
    for (lo, hi), w_ref, b_ref, c_ref, s_ref in (
        (_B2_SLICES[0], wh_ref, bh_ref, ch_ref, sh_ref),
        (_B2_SLICES[1], wd_ref, bd_ref, cd_ref, sd_ref),
        (_B2_SLICES[2], ww_ref, bw_ref, cw_ref, sw_ref),
    ):
        o = (jnp.dot(tx0[:, lo:hi], w_ref[0], preferred_element_type=jnp.float32)
             + jnp.dot(tx1[:, lo:hi], w_ref[1], preferred_element_type=jnp.float32)
             + jnp.dot(tx2[:, lo:hi], w_ref[2], preferred_element_type=jnp.float32))
        o = jnp.maximum(o + b_ref[...], 0.0)
        o = _tconv_relu(o, c_ref, 32)
        p = jnp.maximum(
            jnp.dot(o, wl_ref[...], preferred_element_type=jnp.float32)
            + bl_ref[...], 0.0)
        res = res + s_ref[...] * p
    out_ref[...] = res


def _block2_hop2(tx1, y1, dinv, adiag, A, wh, wd, ww, bh, bd, bw,
                 ch, cd, cw, wl, bl, sh, sd, sw):
    full = lambda s: pl.BlockSpec(s, lambda i: (0,) * len(s))
    return pl.pallas_call(
        _block2_hop2_body,
        grid=(_GRID,),
        in_specs=[
            full((_N, 768)),
            pl.BlockSpec((_TILE, 768), lambda i: (i, 0)),
            full((_N, 1)),
            full((_N, 1)),
            pl.BlockSpec((_TILE, _N), lambda i: (i, 0)),
            full((3, 256, 128)), full((3, 256, 128)), full((3, 256, 128)),
            full((1, 128)), full((1, 128)), full((1, 128)),
            full((1, 4)), full((1, 4)), full((1, 4)),
            full((128, 48)), full((1, 48)),
            full((1, 48)), full((1, 48)), full((1, 48)),
        ],
        out_specs=pl.BlockSpec((_TILE, 48), lambda i: (i, 0)),
        out_shape=jax.ShapeDtypeStruct((_N, 48), jnp.float32),
        scratch_shapes=[pltpu.VMEM((_N, 768), jnp.float32)],
    )(tx1, y1, dinv, adiag, A, wh, wd, ww, bh, bd, bw, ch, cd, cw,
      wl, bl, sh, sd, sw)


def kernel(Xh, Xd, Xw, A, WgH1, bgH1, wcH1, bcH1, WgH2, bgH2, wcH2, bcH2,
           WgD1, bgD1, wcD1, bcD1, WgD2, bgD2, wcD2, bcD2,
           WgW1, bgW1, wcW1, bcW1, WgW2, bgW2, wcW2, bcW2,
           WlD, blD, Wh, Wd, Ww):
    eye = jnp.eye(_B, dtype=jnp.float32)

    def cat_bn(X):
        return X[:, :, 0, :].transpose(1, 0, 2).reshape(_N, -1)

    def bdiag(Wg):
        return jnp.stack([jnp.kron(eye, Wg[k]) for k in range(3)])

    def brow(b):
        return jnp.tile(b, _B)[None, :]

    def cpack(wc, bc):
        return jnp.concatenate([wc.reshape(3), bc.reshape(1)])[None, :]

    xc = jnp.concatenate([cat_bn(Xh), cat_bn(Xd), cat_bn(Xw)], axis=1)

    dinv, adiag = _rowstats(A)
    tx1 = _lhop1(xc, dinv, adiag, A)
    y1 = _block1_hop2(tx1, xc, dinv, adiag, A,
                      bdiag(WgH1), bdiag(WgD1), bdiag(WgW1),
                      brow(bgH1), brow(bgD1), brow(bgW1),
                      cpack(wcH1, bcH1), cpack(wcD1, bcD1), cpack(wcW1, bcW1))
    tx1b = _lhop1(y1, dinv, adiag, A)
    out48 = _block2_hop2(tx1b, y1, dinv, adiag, A,
                         bdiag(WgH2), bdiag(WgD2), bdiag(WgW2),
                         brow(bgH2), brow(bgD2), brow(bgW2),
                         cpack(wcH2, bcH2), cpack(wcD2, bcD2),
                         cpack(wcW2, bcW2),
                         jnp.kron(eye, WlD.T), brow(blD),
                         brow(Wh), brow(Wd), brow(Ww))
    return out48.reshape(_N, _B, 12).transpose(1, 0, 2)[:, :, None, :]


# speedup vs baseline: 2.2297x; 1.3576x over previous
import jax
import jax.numpy as jnp
from jax import lax
from jax.experimental import pallas as pl
from jax.experimental.pallas import tpu as pltpu

_N = 2048
_B = 4
_TILE = 256
_GRID = _N // _TILE


def _prep_body(a_ref, dinv_ref, adiag_ref, abf_ref):
    i = pl.program_id(0)
    a = a_ref[...]
    rowsum = jnp.sum(a, axis=1)
    row = lax.broadcasted_iota(jnp.int32, a.shape, 0)
    col = lax.broadcasted_iota(jnp.int32, a.shape, 1)
    diag = jnp.sum(jnp.where(col == row + i * _TILE, a, 0.0), axis=1)
    deg = rowsum - diag
    pos = deg > 0.0
    dinv = jnp.where(pos, lax.rsqrt(jnp.where(pos, deg, 1.0)), 0.0)
    dinv_ref[...] = dinv[:, None]
    adiag_ref[...] = diag[:, None]
    abf_ref[...] = a.astype(jnp.bfloat16)


def _prep(A):
    return pl.pallas_call(
        _prep_body,
        grid=(_GRID,),
        in_specs=[pl.BlockSpec((_TILE, _N), lambda i: (i, 0))],
        out_specs=(pl.BlockSpec((_TILE, 1), lambda i: (i, 0)),
                   pl.BlockSpec((_TILE, 1), lambda i: (i, 0)),
                   pl.BlockSpec((_TILE, _N), lambda i: (i, 0))),
        out_shape=(jax.ShapeDtypeStruct((_N, 1), jnp.float32),
                   jax.ShapeDtypeStruct((_N, 1), jnp.float32),
                   jax.ShapeDtypeStruct((_N, _N), jnp.bfloat16)),
    )(A)


def _tconv_relu(x, wc_ref, group):
    t, _ = x.shape
    z = jnp.zeros((t, 1), dtype=x.dtype)
    xl = jnp.concatenate([z, x[:, :-1]], axis=1)
    xr = jnp.concatenate([x[:, 1:], z], axis=1)
    colm = lax.broadcasted_iota(jnp.int32, x.shape, 1) % group
    xl = jnp.where(colm == 0, 0.0, xl)
    xr = jnp.where(colm == group - 1, 0.0, xr)
    y = (wc_ref[:, 0:1] * xl + wc_ref[:, 1:2] * x + wc_ref[:, 2:3] * xr
         + wc_ref[:, 3:4])
    return jnp.maximum(y, 0.0)


_B1_SLICES = ((0, 96), (96, 144), (144, 240))
_B2_SLICES = ((0, 256), (256, 512), (512, 768))


def _mega_body(abf_ref, xc_ref, dinv_ref, adiag_ref,
               w1h_ref, w1d_ref, w1w_ref, b1h_ref, b1d_ref, b1w_ref,
               c1h_ref, c1d_ref, c1w_ref,
               w2h_ref, w2d_ref, w2w_ref, b2h_ref, b2d_ref, b2w_ref,
               c2h_ref, c2d_ref, c2w_ref,
               wl_ref, bl_ref, sh_ref, sd_ref, sw_ref,
               out_ref):
    a = abf_ref[...]
    dinv = dinv_ref[...]
    adiag = adiag_ref[...]

    def lhop(xf):
        yf = xf * dinv
        acc = jnp.dot(a, yf.astype(jnp.bfloat16),
                      preferred_element_type=jnp.float32)
        return -dinv * (acc - adiag * yf)

    def cheb(tx0, tx1, tx2, lo, hi, w_ref, b_ref):
        o = (jnp.dot(tx0[:, lo:hi].astype(jnp.bfloat16), w_ref[0],
                     preferred_element_type=jnp.float32)
             + jnp.dot(tx1[:, lo:hi].astype(jnp.bfloat16), w_ref[1],
                       preferred_element_type=jnp.float32)
             + jnp.dot(tx2[:, lo:hi].astype(jnp.bfloat16), w_ref[2],
                       preferred_element_type=jnp.float32))
        return jnp.maximum(o + b_ref[...], 0.0)

    tx0 = xc_ref[...]
    tx1 = lhop(tx0)
    tx2 = 2.0 * lhop(tx1) - tx0
    y1 = jnp.concatenate(
        [_tconv_relu(cheb(tx0, tx1, tx2, lo, hi, w_ref, b_ref), c_ref, 64)
         for (lo, hi), w_ref, b_ref, c_ref in (
             (_B1_SLICES[0], w1h_ref, b1h_ref, c1h_ref),
             (_B1_SLICES[1], w1d_ref, b1d_ref, c1d_ref),
             (_B1_SLICES[2], w1w_ref, b1w_ref, c1w_ref))],
        axis=1)

    tx1b = lhop(y1)
    tx2b = 2.0 * lhop(tx1b) - y1
    res = jnp.zeros((_N, _B * 12), jnp.float32)
    for (lo, hi), w_ref, b_ref, c_ref, s_ref in (
        (_B2_SLICES[0], w2h_ref, b2h_ref, c2h_ref, sh_ref),
        (_B2_SLICES[1], w2d_ref, b2d_ref, c2d_ref, sd_ref),
        (_B2_SLICES[2], w2w_ref, b2w_ref, c2w_ref, sw_ref),
    ):
        o = _tconv_relu(cheb(y1, tx1b, tx2b, lo, hi, w_ref, b_ref),
                        c_ref, 32)
        p = jnp.maximum(
            jnp.dot(o, wl_ref[...], preferred_element_type=jnp.float32)
            + bl_ref[...], 0.0)
        res = res + s_ref[...] * p
    out_ref[...] = res


def _mega(abf, xc, dinv, adiag, w1, b1, c1, w2, b2, c2, wl, bl, ss):
    args = [abf, xc, dinv, adiag, *w1, *b1, *c1, *w2, *b2, *c2, wl, bl, *ss]
    return pl.pallas_call(
        _mega_body,
        out_shape=jax.ShapeDtypeStruct((_N, _B * 12), jnp.float32),
    )(*args)


def kernel(Xh, Xd, Xw, A, WgH1, bgH1, wcH1, bcH1, WgH2, bgH2, wcH2, bcH2,
           WgD1, bgD1, wcD1, bcD1, WgD2, bgD2, wcD2, bcD2,
           WgW1, bgW1, wcW1, bcW1, WgW2, bgW2, wcW2, bcW2,
           WlD, blD, Wh, Wd, Ww):
    eye = jnp.eye(_B, dtype=jnp.float32)

    def cat_bn(X):
        return X[:, :, 0, :].transpose(1, 0, 2).reshape(_N, -1)

    def bdiag(Wg):
        return jnp.stack([jnp.kron(eye, Wg[k]) for k in range(3)]
                         ).astype(jnp.bfloat16)

    def brow(b):
        return jnp.tile(b, _B)[None, :]

    def cpack(wc, bc):
        return jnp.concatenate([wc.reshape(3), bc.reshape(1)])[None, :]

    xc = jnp.concatenate([cat_bn(Xh), cat_bn(Xd), cat_bn(Xw)], axis=1)

    dinv, adiag, abf = _prep(A)
    out48 = _mega(
        abf, xc, dinv, adiag,
        (bdiag(WgH1), bdiag(WgD1), bdiag(WgW1)),
        (brow(bgH1), brow(bgD1), brow(bgW1)),
        (cpack(wcH1, bcH1), cpack(wcD1, bcD1), cpack(wcW1, bcW1)),
        (bdiag(WgH2), bdiag(WgD2), bdiag(WgW2)),
        (brow(bgH2), brow(bgD2), brow(bgW2)),
        (cpack(wcH2, bcH2), cpack(wcD2, bcD2), cpack(wcW2, bcW2)),
        jnp.kron(eye, WlD.T), brow(blD),
        (brow(Wh), brow(Wd), brow(Ww)))
    return out48.reshape(_N, _B, 12).transpose(1, 0, 2)[:, :, None, :]
